# baseline (speedup 1.0000x reference)
import functools

import jax
import jax.numpy as jnp
from jax import lax
from jax.experimental import pallas as pl
from jax.experimental.pallas import tpu as pltpu
from jax.experimental.pallas import tpu_sc as plsc

N_TOKENS = 32768
D_MODEL = 2048
N_EXPERTS = 8
TOP_K = 2

NC = 2
NS = 16
L = 16
NW = NC * NS
CHUNK = N_TOKENS // NW
STEPS = CHUNK // L

TOK_TILE = 512
N_STEPS = N_TOKENS // TOK_TILE
NBUF = 6
LOOKAHEAD = 5


def _copy_slot(x_hbm, xbuf, sems, step, s):
    return pltpu.make_async_copy(
        x_hbm.at[pl.ds(step * TOK_TILE, TOK_TILE), :],
        xbuf.at[s],
        sems.at[s],
    )


def _matmul_body(x_hbm, w_ref, b_ref, out_ref, xbuf, sems):
    i = pl.program_id(0)

    def start(step):
        for s in range(NBUF):
            @pl.when(lax.rem(step, NBUF) == s)
            def _(s=s):
                _copy_slot(x_hbm, xbuf, sems, step, s).start()

    @pl.when(i == 0)
    def _():
        for d in range(LOOKAHEAD):
            _copy_slot(x_hbm, xbuf, sems, d, d).start()

    @pl.when(i + LOOKAHEAD < N_STEPS)
    def _():
        start(i + LOOKAHEAD)

    for s in range(NBUF):
        @pl.when(lax.rem(i, NBUF) == s)
        def _(s=s):
            _copy_slot(x_hbm, xbuf, sems, i, s).wait()
            out_ref[...] = b_ref[...] + jnp.dot(
                xbuf[s], w_ref[...], preferred_element_type=jnp.float32)


def _matmul(x, w, b2d):
    return pl.pallas_call(
        _matmul_body,
        grid=(N_STEPS,),
        in_specs=[
            pl.BlockSpec(memory_space=pl.ANY),
            pl.BlockSpec((D_MODEL, N_EXPERTS), lambda i: (0, 0)),
            pl.BlockSpec((1, N_EXPERTS), lambda i: (0, 0)),
        ],
        out_specs=pl.BlockSpec((TOK_TILE, N_EXPERTS), lambda i: (i, 0)),
        out_shape=jax.ShapeDtypeStruct((N_TOKENS, N_EXPERTS), jnp.float32),
        scratch_shapes=[
            pltpu.VMEM((NBUF, TOK_TILE, D_MODEL), jnp.float32),
            pltpu.SemaphoreType.DMA((NBUF,)),
        ],
        compiler_params=pltpu.CompilerParams(
            dimension_semantics=("arbitrary",),
        ),
    )(x, w, b2d)


def _router_body(logits_hbm, gates_hbm, scores_hbm, idxs_hbm,
                 logits_v, gates_v, scores_v, idxs_v):
    wid = lax.axis_index("s") * NC + lax.axis_index("c")
    pltpu.sync_copy(
        logits_hbm.at[pl.ds(wid * CHUNK * N_EXPERTS, CHUNK * N_EXPERTS)],
        logits_v)

    iota = lax.iota(jnp.int32, L)
    iota_e = iota * N_EXPERTS
    iota_k = iota * TOP_K

    def step(j, carry):
        lbase = iota_e + j * (L * N_EXPERTS)
        i1 = jnp.zeros((L,), jnp.int32)
        m1 = plsc.load_gather(logits_v, [lbase])
        m2 = jnp.full((L,), -jnp.inf, jnp.float32)
        i2 = jnp.zeros((L,), jnp.int32)
        for e in range(1, N_EXPERTS):
            col = jnp.full((L,), e, jnp.int32)
            v = plsc.load_gather(logits_v, [lbase + e])
            gt1 = v > m1
            gt2 = v > m2
            m2 = jnp.where(gt1, m1, jnp.where(gt2, v, m2))
            i2 = jnp.where(gt1, i1, jnp.where(gt2, col, i2))
            m1 = jnp.where(gt1, v, m1)
            i1 = jnp.where(gt1, col, i1)
        r = jnp.exp(m2 - m1)
        g1 = 1.0 / (1.0 + r)
        g2 = r * g1
        obase = iota_k + j * (L * TOP_K)
        plsc.store_scatter(scores_v, [obase], m1)
        plsc.store_scatter(scores_v, [obase + 1], m2)
        plsc.store_scatter(gates_v, [obase], g1)
        plsc.store_scatter(gates_v, [obase + 1], g2)
        plsc.store_scatter(idxs_v, [obase], i1)
        plsc.store_scatter(idxs_v, [obase + 1], i2)
        return carry

    lax.fori_loop(0, STEPS, step, 0)
    obase = wid * CHUNK * TOP_K
    pltpu.sync_copy(gates_v, gates_hbm.at[pl.ds(obase, CHUNK * TOP_K)])
    pltpu.sync_copy(scores_v, scores_hbm.at[pl.ds(obase, CHUNK * TOP_K)])
    pltpu.sync_copy(idxs_v, idxs_hbm.at[pl.ds(obase, CHUNK * TOP_K)])


_router = functools.partial(
    pl.kernel,
    out_type=(
        jax.ShapeDtypeStruct((N_TOKENS * TOP_K,), jnp.float32),
        jax.ShapeDtypeStruct((N_TOKENS * TOP_K,), jnp.float32),
        jax.ShapeDtypeStruct((N_TOKENS * TOP_K,), jnp.int32),
    ),
    mesh=plsc.VectorSubcoreMesh(
        core_axis_name="c", subcore_axis_name="s",
        num_cores=NC, num_subcores=NS,
    ),
    scratch_types=[
        pltpu.VMEM((CHUNK * N_EXPERTS,), jnp.float32),
        pltpu.VMEM((CHUNK * TOP_K,), jnp.float32),
        pltpu.VMEM((CHUNK * TOP_K,), jnp.float32),
        pltpu.VMEM((CHUNK * TOP_K,), jnp.int32),
    ],
    compiler_params=pltpu.CompilerParams(needs_layout_passes=False),
)(_router_body)


def kernel(x, W, b):
    logits = _matmul(x, W, b.reshape(1, N_EXPERTS))
    gates, top_scores, top_idxs = _router(logits.reshape(-1))
    shape = (N_TOKENS, TOP_K)
    return (gates.reshape(shape), top_scores.reshape(shape),
            top_idxs.reshape(shape))


# speedup vs baseline: 1.8604x; 1.8604x over previous
import functools

import jax
import jax.numpy as jnp
from jax import lax
from jax.experimental import pallas as pl
from jax.experimental.pallas import tpu as pltpu
from jax.experimental.pallas import tpu_sc as plsc

N_TOKENS = 32768
D_MODEL = 2048
N_EXPERTS = 8
TOP_K = 2

NC = 2
NS = 16
L = 16
NW = NC * NS
CHUNK = N_TOKENS // NW
STEPS = CHUNK // L
BLK = 128

TOK_TILE = 512
N_STEPS = N_TOKENS // TOK_TILE
NBUF = 6
LOOKAHEAD = 5


def _copy_slot(x_hbm, xbuf, sems, step, s):
    return pltpu.make_async_copy(
        x_hbm.at[pl.ds(step * TOK_TILE, TOK_TILE), :],
        xbuf.at[s],
        sems.at[s],
    )


def _matmul_body(x_hbm, w_ref, b_ref, out_ref, xbuf, sems):
    i = pl.program_id(0)

    def start(step):
        for s in range(NBUF):
            @pl.when(lax.rem(step, NBUF) == s)
            def _(s=s):
                _copy_slot(x_hbm, xbuf, sems, step, s).start()

    @pl.when(i == 0)
    def _():
        for d in range(LOOKAHEAD):
            _copy_slot(x_hbm, xbuf, sems, d, d).start()

    @pl.when(i + LOOKAHEAD < N_STEPS)
    def _():
        start(i + LOOKAHEAD)

    for s in range(NBUF):
        @pl.when(lax.rem(i, NBUF) == s)
        def _(s=s):
            _copy_slot(x_hbm, xbuf, sems, i, s).wait()
            out_ref[...] = b_ref[...] + lax.dot_general(
                w_ref[...], xbuf[s],
                dimension_numbers=(((0,), (1,)), ((), ())),
                preferred_element_type=jnp.float32,
            )


def _matmul(x, w, b2d):
    return pl.pallas_call(
        _matmul_body,
        grid=(N_STEPS,),
        in_specs=[
            pl.BlockSpec(memory_space=pl.ANY),
            pl.BlockSpec((D_MODEL, N_EXPERTS), lambda i: (0, 0)),
            pl.BlockSpec((N_EXPERTS, 1), lambda i: (0, 0)),
        ],
        out_specs=pl.BlockSpec((N_EXPERTS, TOK_TILE), lambda i: (0, i)),
        out_shape=jax.ShapeDtypeStruct((N_EXPERTS, N_TOKENS), jnp.float32),
        scratch_shapes=[
            pltpu.VMEM((NBUF, TOK_TILE, D_MODEL), jnp.float32),
            pltpu.SemaphoreType.DMA((NBUF,)),
        ],
        compiler_params=pltpu.CompilerParams(
            dimension_semantics=("arbitrary",),
        ),
    )(x, w, b2d)


def _router_body(logits_hbm, gates_hbm, scores_hbm, idxs_hbm,
                 le_v, g_v, s_v, i_v):
    wid = lax.axis_index("s") * NC + lax.axis_index("c")
    base = wid * CHUNK
    for e in range(N_EXPERTS):
        pltpu.sync_copy(logits_hbm.at[e, pl.ds(base, CHUNK)],
                        le_v.at[pl.ds(e * CHUNK, CHUNK)])

    def step(j, carry):
        off = j * L
        i1 = jnp.zeros((L,), jnp.int32)
        m1 = le_v[pl.ds(off, L)]
        m2 = jnp.full((L,), -jnp.inf, jnp.float32)
        i2 = jnp.zeros((L,), jnp.int32)
        for e in range(1, N_EXPERTS):
            col = jnp.full((L,), e, jnp.int32)
            v = le_v[pl.ds(e * CHUNK + off, L)]
            gt1 = v > m1
            gt2 = v > m2
            m2 = jnp.where(gt1, m1, jnp.where(gt2, v, m2))
            i2 = jnp.where(gt1, i1, jnp.where(gt2, col, i2))
            m1 = jnp.where(gt1, v, m1)
            i1 = jnp.where(gt1, col, i1)
        r = jnp.exp(m2 - m1)
        g1 = 1.0 / (1.0 + r)
        g2 = r * g1
        d0 = (j // (BLK // L)) * (TOP_K * BLK) + (j % (BLK // L)) * L
        d1 = d0 + BLK
        s_v[pl.ds(d0, L)] = m1
        s_v[pl.ds(d1, L)] = m2
        g_v[pl.ds(d0, L)] = g1
        g_v[pl.ds(d1, L)] = g2
        i_v[pl.ds(d0, L)] = i1
        i_v[pl.ds(d1, L)] = i2
        return carry

    lax.fori_loop(0, STEPS, step, 0)
    obase = base * TOP_K
    pltpu.sync_copy(g_v, gates_hbm.at[pl.ds(obase, CHUNK * TOP_K)])
    pltpu.sync_copy(s_v, scores_hbm.at[pl.ds(obase, CHUNK * TOP_K)])
    pltpu.sync_copy(i_v, idxs_hbm.at[pl.ds(obase, CHUNK * TOP_K)])


_router = functools.partial(
    pl.kernel,
    out_type=(
        jax.ShapeDtypeStruct((N_TOKENS * TOP_K,), jnp.float32),
        jax.ShapeDtypeStruct((N_TOKENS * TOP_K,), jnp.float32),
        jax.ShapeDtypeStruct((N_TOKENS * TOP_K,), jnp.int32),
    ),
    mesh=plsc.VectorSubcoreMesh(
        core_axis_name="c", subcore_axis_name="s",
        num_cores=NC, num_subcores=NS,
    ),
    scratch_types=[
        pltpu.VMEM((CHUNK * N_EXPERTS,), jnp.float32),
        pltpu.VMEM((CHUNK * TOP_K,), jnp.float32),
        pltpu.VMEM((CHUNK * TOP_K,), jnp.float32),
        pltpu.VMEM((CHUNK * TOP_K,), jnp.int32),
    ],
    compiler_params=pltpu.CompilerParams(needs_layout_passes=False),
)(_router_body)


def _detile(o):
    return (o.reshape(N_TOKENS // BLK, TOP_K, BLK)
             .transpose(0, 2, 1)
             .reshape(N_TOKENS, TOP_K))


def kernel(x, W, b):
    logits_t = _matmul(x, W, b.reshape(N_EXPERTS, 1))
    gates, top_scores, top_idxs = _router(logits_t)
    return (_detile(gates), _detile(top_scores), _detile(top_idxs))
